# Initial kernel scaffold; baseline (speedup 1.0000x reference)
import jax
import jax.numpy as jnp
from jax.experimental import pallas as pl


def kernel(x, edge_index, batch, W1, b1, W2, b2, W3, b3):
    raise NotImplementedError("write your pallas kernel here")


# speedup vs baseline: 31.4077x; 31.4077x over previous
import functools

import jax
import jax.numpy as jnp
from jax import lax
from jax.experimental import pallas as pl
from jax.experimental.pallas import tpu as pltpu
from jax.experimental.pallas import tpu_sc as plsc

N = 10000
E = 320000
D = 128
NS = 16
EPT = E // NS
ROWS = (EPT + 127) // 128
EP = NS * ROWS * 128
CHUNK = 632
NPAD = NS * CHUNK

_f32 = jnp.float32


def _sc_uchain(srcp, dstp, ones):
    mesh = plsc.VectorSubcoreMesh(core_axis_name="c", subcore_axis_name="s")

    @functools.partial(
        pl.kernel,
        out_type=[jax.ShapeDtypeStruct((NPAD,), _f32)] * 3,
        mesh=mesh,
        scratch_types=[
            pltpu.VMEM((ROWS, 128), jnp.int32),
            pltpu.VMEM((ROWS, 128), jnp.int32),
            pltpu.VMEM((128,), _f32),
            pltpu.VMEM((128,), _f32),
            pltpu.VMEM((CHUNK,), _f32),
            pltpu.VMEM_SHARED((NPAD,), _f32),
            pltpu.VMEM_SHARED((NPAD,), _f32),
            pltpu.VMEM_SHARED((NPAD,), _f32),
        ],
    )
    def k(srcp_hbm, dstp_hbm, ones_hbm, u1_hbm, u2_hbm, u3_hbm,
          src_v, dst_v, vals_v, ones_v, tmp_v, acc3, acc2, acc1):
        c = lax.axis_index("c")
        s = lax.axis_index("s")
        sl = pl.ds(s * CHUNK, CHUNK)

        pltpu.sync_copy(srcp_hbm.at[s], src_v)
        pltpu.sync_copy(dstp_hbm.at[s], dst_v)
        pltpu.sync_copy(ones_hbm.at[pl.ds(0, 128)], ones_v)

        @pl.when(s == 0)
        def _():
            pltpu.sync_copy(ones_hbm, acc3)

        plsc.subcore_barrier()

        @pl.loop(0, ROWS)
        def _(j):
            pltpu.sync_copy(ones_v, acc3.at[src_v.at[j]], add=True)

        plsc.subcore_barrier()

        pltpu.sync_copy(acc3.at[sl], tmp_v)
        pltpu.sync_copy(tmp_v, acc2.at[sl])
        plsc.subcore_barrier()

        @pl.loop(0, ROWS)
        def _(j):
            pltpu.sync_copy(acc3.at[dst_v.at[j]], vals_v)
            pltpu.sync_copy(vals_v, acc2.at[src_v.at[j]], add=True)

        plsc.subcore_barrier()

        pltpu.sync_copy(acc2.at[sl], tmp_v)
        pltpu.sync_copy(tmp_v, acc1.at[sl])
        plsc.subcore_barrier()

        @pl.loop(0, ROWS)
        def _(j):
            pltpu.sync_copy(acc2.at[dst_v.at[j]], vals_v)
            pltpu.sync_copy(vals_v, acc1.at[src_v.at[j]], add=True)

        plsc.subcore_barrier()

        @pl.when(c == 0)
        def _():
            pltpu.sync_copy(acc1.at[sl], tmp_v)
            pltpu.sync_copy(tmp_v, u1_hbm.at[sl])
            pltpu.sync_copy(acc2.at[sl], tmp_v)
            pltpu.sync_copy(tmp_v, u2_hbm.at[sl])
            pltpu.sync_copy(acc3.at[sl], tmp_v)
            pltpu.sync_copy(tmp_v, u3_hbm.at[sl])

    return k(srcp, dstp, ones)


def _tc_tail(u1, u2, u3, x, W1, b1, W2, b2, W3, b3):
    def body(u1_ref, u2_ref, u3_ref, x_ref, W1_ref, b1_ref, W2_ref, b2_ref,
             W3_ref, b3_ref, out_ref):
        hi = lax.Precision.HIGHEST
        u1v = u1_ref[...]
        r = lax.dot_general(u1v, x_ref[...], (((1,), (0,)), ((), ())),
                            precision=hi, preferred_element_type=_f32)
        s1 = jnp.sum(u1v)
        s2 = jnp.sum(u2_ref[...])
        s3 = jnp.sum(u3_ref[...])
        t = lax.dot_general(r, W1_ref[...], (((1,), (1,)), ((), ())),
                            precision=hi, preferred_element_type=_f32)
        t = t + s1 * b1_ref[...]
        t = lax.dot_general(t, W2_ref[...], (((1,), (1,)), ((), ())),
                            precision=hi, preferred_element_type=_f32)
        t = t + s2 * b2_ref[...]
        t = lax.dot_general(t, W3_ref[...], (((1,), (1,)), ((), ())),
                            precision=hi, preferred_element_type=_f32)
        t = t + s3 * b3_ref[...]
        out_ref[...] = t * (1.0 / N)

    return pl.pallas_call(
        body,
        out_shape=jax.ShapeDtypeStruct((1, D), _f32),
    )(u1, u2, u3, x, W1, b1, W2, b2, W3, b3)


def kernel(x, edge_index, batch, W1, b1, W2, b2, W3, b3):
    src = edge_index[0]
    dst = edge_index[1]
    pad = jnp.full((EP - E,), N, dtype=jnp.int32)
    srcp = jnp.concatenate([src, pad]).reshape(NS, ROWS, 128)
    dstp = jnp.concatenate([dst, pad]).reshape(NS, ROWS, 128)
    ones = jnp.ones((NPAD,), dtype=_f32)

    u1p, u2p, u3p = _sc_uchain(srcp, dstp, ones)
    u1 = u1p[:N].reshape(1, N)
    u2 = u2p[:N].reshape(1, N)
    u3 = u3p[:N].reshape(1, N)

    return _tc_tail(u1, u2, u3, x, W1, b1.reshape(1, D),
                    W2, b2.reshape(1, D), W3, b3.reshape(1, D))


# speedup vs baseline: 43.0337x; 1.3702x over previous
import functools

import jax
import jax.numpy as jnp
from jax import lax
from jax.experimental import pallas as pl
from jax.experimental.pallas import tpu as pltpu
from jax.experimental.pallas import tpu_sc as plsc

N = 10000
E = 320000
D = 128
NC = 2
NS = 16
NW = NC * NS
RW = 79
EP = NW * RW * 128
CHUNK = 640
NPAD = NS * CHUNK

_f32 = jnp.float32
_mesh = lambda: plsc.VectorSubcoreMesh(core_axis_name="c", subcore_axis_name="s")


def _fill(ref, value):
    v = jnp.full((16,), value, _f32)

    @pl.loop(0, ref.shape[0], step=16)
    def _(i):
        ref[pl.ds(i, 16)] = v


def _sc_bincount(srcp):
    @functools.partial(
        pl.kernel,
        out_type=jax.ShapeDtypeStruct((NC, NPAD), _f32),
        mesh=_mesh(),
        scratch_types=[
            pltpu.VMEM((RW, 128), jnp.int32),
            pltpu.VMEM((128,), _f32),
            pltpu.VMEM((CHUNK,), _f32),
            pltpu.VMEM_SHARED((NPAD,), _f32),
        ],
    )
    def k(srcp_hbm, p_hbm, src_v, ones_v, tmp_v, acc):
        c = lax.axis_index("c")
        s = lax.axis_index("s")
        w = c * NS + s
        sl = pl.ds(s * CHUNK, CHUNK)

        pltpu.sync_copy(srcp_hbm.at[w], src_v)
        _fill(ones_v, 1.0)
        _fill(tmp_v, 0.0)
        pltpu.sync_copy(tmp_v, acc.at[sl])
        plsc.subcore_barrier()

        @pl.loop(0, RW)
        def _(j):
            pltpu.sync_copy(ones_v, acc.at[src_v.at[j]], add=True)

        plsc.subcore_barrier()
        pltpu.sync_copy(acc.at[sl], tmp_v)
        pltpu.sync_copy(tmp_v, p_hbm.at[c].at[sl])

    return k(srcp)


def _sc_round(p_prev, srcp, dstp, plus_one):
    @functools.partial(
        pl.kernel,
        out_type=jax.ShapeDtypeStruct((NC, NPAD), _f32),
        mesh=_mesh(),
        scratch_types=[
            pltpu.VMEM((RW, 128), jnp.int32),
            pltpu.VMEM((RW, 128), jnp.int32),
            pltpu.VMEM((128,), _f32),
            pltpu.VMEM((CHUNK,), _f32),
            pltpu.VMEM((CHUNK,), _f32),
            pltpu.VMEM_SHARED((NPAD,), _f32),
            pltpu.VMEM_SHARED((NPAD,), _f32),
        ],
    )
    def k(p_hbm, srcp_hbm, dstp_hbm, out_hbm,
          src_v, dst_v, vals_v, tmp_v, tmp2_v, acc_prev, acc_new):
        c = lax.axis_index("c")
        s = lax.axis_index("s")
        w = c * NS + s
        sl = pl.ds(s * CHUNK, CHUNK)

        pltpu.sync_copy(srcp_hbm.at[w], src_v)
        pltpu.sync_copy(dstp_hbm.at[w], dst_v)

        pltpu.sync_copy(p_hbm.at[0].at[sl], tmp_v)
        pltpu.sync_copy(p_hbm.at[1].at[sl], tmp2_v)
        bias = jnp.full((16,), 1.0 if plus_one else 0.0, _f32)

        @pl.loop(0, CHUNK, step=16)
        def _(i):
            tmp_v[pl.ds(i, 16)] = (tmp_v[pl.ds(i, 16)] + tmp2_v[pl.ds(i, 16)]
                                   + bias)

        pltpu.sync_copy(tmp_v, acc_prev.at[sl])

        @pl.when(c == 0)
        def _():
            pltpu.sync_copy(tmp_v, acc_new.at[sl])

        @pl.when(c != 0)
        def _():
            _fill(tmp2_v, 0.0)
            pltpu.sync_copy(tmp2_v, acc_new.at[sl])

        plsc.subcore_barrier()

        @pl.loop(0, RW)
        def _(j):
            pltpu.sync_copy(acc_prev.at[dst_v.at[j]], vals_v)
            pltpu.sync_copy(vals_v, acc_new.at[src_v.at[j]], add=True)

        plsc.subcore_barrier()
        pltpu.sync_copy(acc_new.at[sl], tmp_v)
        pltpu.sync_copy(tmp_v, out_hbm.at[c].at[sl])

    return k(p_prev, srcp, dstp)


def _tc_tail(p1, p2, p3, x, W1, b1, W2, b2, W3, b3):
    def body(p1_ref, p2_ref, p3_ref, x_ref, W1_ref, b1_ref, W2_ref, b2_ref,
             W3_ref, b3_ref, out_ref):
        hi = lax.Precision.HIGHEST
        u1v = p1_ref[0:1, :] + p1_ref[1:2, :]
        r = lax.dot_general(u1v, x_ref[...], (((1,), (0,)), ((), ())),
                            precision=hi, preferred_element_type=_f32)
        s1 = jnp.sum(u1v)
        s2 = jnp.sum(p2_ref[...])
        s3 = jnp.sum(p3_ref[...]) + float(N)
        t = lax.dot_general(r, W1_ref[...], (((1,), (1,)), ((), ())),
                            precision=hi, preferred_element_type=_f32)
        t = t + s1 * b1_ref[...]
        t = lax.dot_general(t, W2_ref[...], (((1,), (1,)), ((), ())),
                            precision=hi, preferred_element_type=_f32)
        t = t + s2 * b2_ref[...]
        t = lax.dot_general(t, W3_ref[...], (((1,), (1,)), ((), ())),
                            precision=hi, preferred_element_type=_f32)
        t = t + s3 * b3_ref[...]
        out_ref[...] = t * (1.0 / N)

    return pl.pallas_call(
        body,
        out_shape=jax.ShapeDtypeStruct((1, D), _f32),
    )(p1, p2, p3, x, W1, b1, W2, b2, W3, b3)


def kernel(x, edge_index, batch, W1, b1, W2, b2, W3, b3):
    src = edge_index[0]
    dst = edge_index[1]
    pad = jnp.full((EP - E,), N, dtype=jnp.int32)
    srcp = jnp.concatenate([src, pad]).reshape(NW, RW, 128)
    dstp = jnp.concatenate([dst, pad]).reshape(NW, RW, 128)

    p3 = _sc_bincount(srcp)
    p2 = _sc_round(p3, srcp, dstp, plus_one=True)
    p1 = _sc_round(p2, srcp, dstp, plus_one=False)

    return _tc_tail(p1[:, :N], p2[:, :N], p3[:, :N], x, W1, b1.reshape(1, D),
                    W2, b2.reshape(1, D), W3, b3.reshape(1, D))


# speedup vs baseline: 54.4729x; 1.2658x over previous
import functools

import jax
import jax.numpy as jnp
from jax import lax
from jax.experimental import pallas as pl
from jax.experimental.pallas import tpu as pltpu
from jax.experimental.pallas import tpu_sc as plsc

N = 10000
E = 320000
D = 128
NC = 2
NS = 16
NW = NC * NS
RW = 79
EP = NW * RW * 128
CHUNK = 640
NPAD = NS * CHUNK

_f32 = jnp.float32
_mesh = lambda: plsc.VectorSubcoreMesh(core_axis_name="c", subcore_axis_name="s")


def _fill(ref, value):
    v = jnp.full((16,), value, _f32)

    @pl.loop(0, ref.shape[0], step=16)
    def _(i):
        ref[pl.ds(i, 16)] = v


def _sc_bincount(srcr):
    @functools.partial(
        pl.kernel,
        out_type=jax.ShapeDtypeStruct((NC, NPAD), _f32),
        mesh=_mesh(),
        scratch_types=[
            pltpu.VMEM((RW, 128), jnp.int32),
            pltpu.VMEM((128,), _f32),
            pltpu.VMEM((CHUNK,), _f32),
            pltpu.VMEM_SHARED((NPAD,), _f32),
            pltpu.SemaphoreType.DMA,
            pltpu.SemaphoreType.DMA,
            pltpu.SemaphoreType.DMA,
            pltpu.SemaphoreType.DMA,
        ],
    )
    def k(srcr_hbm, p_hbm, src_v, ones_v, tmp_v, acc, *sems):
        c = lax.axis_index("c")
        s = lax.axis_index("s")
        w = c * NS + s
        sl = pl.ds(s * CHUNK, CHUNK)

        pltpu.sync_copy(srcr_hbm.at[w], src_v)
        _fill(ones_v, 1.0)
        _fill(tmp_v, 0.0)
        pltpu.sync_copy(tmp_v, acc.at[sl])
        plsc.subcore_barrier()

        for b in range(4):
            pltpu.async_copy(ones_v, acc.at[src_v.at[b]], sems[b], add=True)

        @pl.loop(0, RW - 3, step=4)
        def _(j):
            for b in range(4):
                jj = j + b
                pltpu.make_async_copy(ones_v, acc.at[src_v.at[jj]],
                                      sems[b]).wait()

                @pl.when(jj + 4 < RW)
                def _():
                    pltpu.async_copy(ones_v, acc.at[src_v.at[jj + 4]],
                                     sems[b], add=True)

        for b in range(RW % 4):
            pltpu.make_async_copy(ones_v, acc.at[src_v.at[RW - RW % 4 + b]],
                                  sems[b]).wait()

        plsc.subcore_barrier()
        pltpu.sync_copy(acc.at[sl], tmp_v)
        pltpu.sync_copy(tmp_v, p_hbm.at[c].at[sl])

    return k(srcr)


def _sc_round(p_prev, srcr, dstr, plus_one):
    @functools.partial(
        pl.kernel,
        out_type=jax.ShapeDtypeStruct((NC, NPAD), _f32),
        mesh=_mesh(),
        scratch_types=[
            pltpu.VMEM((RW, 128), jnp.int32),
            pltpu.VMEM((RW, 128), jnp.int32),
            pltpu.VMEM((128,), _f32),
            pltpu.VMEM((128,), _f32),
            pltpu.VMEM((128,), _f32),
            pltpu.VMEM((128,), _f32),
            pltpu.VMEM((CHUNK,), _f32),
            pltpu.VMEM((CHUNK,), _f32),
            pltpu.VMEM_SHARED((NPAD,), _f32),
            pltpu.VMEM_SHARED((NPAD,), _f32),
            pltpu.SemaphoreType.DMA,
            pltpu.SemaphoreType.DMA,
            pltpu.SemaphoreType.DMA,
            pltpu.SemaphoreType.DMA,
        ],
    )
    def k(p_hbm, srcr_hbm, dstr_hbm, out_hbm,
          src_v, dst_v, v0, v1, v2, v3, tmp_v, tmp2_v, acc_prev, acc_new,
          *sems):
        vals = (v0, v1, v2, v3)
        c = lax.axis_index("c")
        s = lax.axis_index("s")
        w = c * NS + s
        sl = pl.ds(s * CHUNK, CHUNK)

        pltpu.sync_copy(srcr_hbm.at[w], src_v)
        pltpu.sync_copy(dstr_hbm.at[w], dst_v)

        pltpu.sync_copy(p_hbm.at[0].at[sl], tmp_v)
        pltpu.sync_copy(p_hbm.at[1].at[sl], tmp2_v)
        bias = jnp.full((16,), 1.0 if plus_one else 0.0, _f32)

        @pl.loop(0, CHUNK, step=16)
        def _(i):
            tmp_v[pl.ds(i, 16)] = (tmp_v[pl.ds(i, 16)] + tmp2_v[pl.ds(i, 16)]
                                   + bias)

        pltpu.sync_copy(tmp_v, acc_prev.at[sl])

        @pl.when(c == 0)
        def _():
            pltpu.sync_copy(tmp_v, acc_new.at[sl])

        @pl.when(c != 0)
        def _():
            _fill(tmp2_v, 0.0)
            pltpu.sync_copy(tmp2_v, acc_new.at[sl])

        plsc.subcore_barrier()

        for b in range(4):
            pltpu.async_copy(acc_prev.at[dst_v.at[b]], vals[b], sems[b])

        @pl.loop(0, RW - 3, step=4)
        def _(j):
            for b in range(4):
                jj = j + b
                pltpu.make_async_copy(acc_prev.at[dst_v.at[jj]], vals[b],
                                      sems[b]).wait()
                pltpu.sync_copy(vals[b], acc_new.at[src_v.at[jj]], add=True)

                @pl.when(jj + 4 < RW)
                def _():
                    pltpu.async_copy(acc_prev.at[dst_v.at[jj + 4]], vals[b],
                                     sems[b])

        for b in range(RW % 4):
            jj = RW - RW % 4 + b
            pltpu.make_async_copy(acc_prev.at[dst_v.at[jj]], vals[b],
                                  sems[b]).wait()
            pltpu.sync_copy(vals[b], acc_new.at[src_v.at[jj]], add=True)

        plsc.subcore_barrier()
        pltpu.sync_copy(acc_new.at[sl], tmp_v)
        pltpu.sync_copy(tmp_v, out_hbm.at[c].at[sl])

    return k(p_prev, srcr, dstr)


def _tc_tail(p1, p2, p3, x, W1, b1, W2, b2, W3, b3):
    def body(p1_ref, p2_ref, p3_ref, x_ref, W1_ref, b1_ref, W2_ref, b2_ref,
             W3_ref, b3_ref, out_ref):
        hi = lax.Precision.HIGHEST
        u1v = p1_ref[0:1, :] + p1_ref[1:2, :]
        r = lax.dot_general(u1v, x_ref[...], (((1,), (0,)), ((), ())),
                            precision=hi, preferred_element_type=_f32)
        s1 = jnp.sum(u1v)
        s2 = jnp.sum(p2_ref[...])
        s3 = jnp.sum(p3_ref[...]) + float(N)
        t = lax.dot_general(r, W1_ref[...], (((1,), (1,)), ((), ())),
                            precision=hi, preferred_element_type=_f32)
        t = t + s1 * b1_ref[...]
        t = lax.dot_general(t, W2_ref[...], (((1,), (1,)), ((), ())),
                            precision=hi, preferred_element_type=_f32)
        t = t + s2 * b2_ref[...]
        t = lax.dot_general(t, W3_ref[...], (((1,), (1,)), ((), ())),
                            precision=hi, preferred_element_type=_f32)
        t = t + s3 * b3_ref[...]
        out_ref[...] = t * (1.0 / N)

    return pl.pallas_call(
        body,
        out_shape=jax.ShapeDtypeStruct((1, D), _f32),
    )(p1, p2, p3, x, W1, b1, W2, b2, W3, b3)


def kernel(x, edge_index, batch, W1, b1, W2, b2, W3, b3):
    pad = jnp.full((EP - E,), N, dtype=jnp.int32)
    srcr = jnp.concatenate([edge_index[0], pad]).reshape(NW, RW, 128)
    dstr = jnp.concatenate([edge_index[1], pad]).reshape(NW, RW, 128)

    p3 = _sc_bincount(srcr)
    p2 = _sc_round(p3, srcr, dstr, plus_one=True)
    p1 = _sc_round(p2, srcr, dstr, plus_one=False)

    return _tc_tail(p1[:, :N], p2[:, :N], p3[:, :N], x, W1, b1.reshape(1, D),
                    W2, b2.reshape(1, D), W3, b3.reshape(1, D))
